# Initial kernel scaffold; baseline (speedup 1.0000x reference)
import jax
import jax.numpy as jnp
from jax.experimental import pallas as pl


def kernel(xyz, xyz_s, params):
    raise NotImplementedError("write your pallas kernel here")


# speedup vs baseline: 1.0619x; 1.0619x over previous
import jax
import jax.numpy as jnp
from jax.experimental import pallas as pl


def _group_points(fea, idx):
    return jax.vmap(lambda f, i: f[:, i])(fea, idx)


def _knn(a, b, k):
    inner = -2.0 * jnp.einsum('bcn,bcm->bnm', a, b)
    a2 = jnp.sum(a * a, axis=1)[:, :, None]
    b2 = jnp.sum(b * b, axis=1)[:, None, :]
    dis = -a2 - inner - b2
    return jax.lax.top_k(dis, k)[1]


def _get_edge_point(xyz1, xyz2, idx):
    xyz_knn = _group_points(xyz2, idx)
    x = jnp.broadcast_to(xyz1[:, :, :, None], xyz_knn.shape)
    xyz_r = x - xyz_knn
    xyz_d = jnp.sqrt(jnp.sum(xyz_r * xyz_r, axis=1, keepdims=True) + 1e-12)
    return jnp.concatenate([xyz_d, x, xyz_knn, xyz_r], axis=1)


def _get_edge_feature(fea1, fea2, idx):
    fea_knn = _group_points(fea2, idx)
    f = jnp.broadcast_to(fea1[:, :, :, None], fea_knn.shape)
    return jnp.concatenate([f, fea_knn], axis=1)


def _c2d(x, p, act=True):
    y = jnp.einsum('oc,bcnk->bonk', p['w'], x) + p['b'][None, :, None, None]
    return jax.nn.relu(y) if act else y


def _c1d_pallas(x, p, act=True):
    B, C, N = x.shape
    O = p['w'].shape[0]
    w = p['w']
    b = p['b']

    def body(x_ref, w_ref, b_ref, o_ref):
        y = jnp.dot(w_ref[...], x_ref[0], preferred_element_type=jnp.float32)
        y = y + b_ref[...][:, None]
        if act:
            y = jnp.maximum(y, 0.0)
        o_ref[0] = y

    BN = 512
    out = pl.pallas_call(
        body,
        grid=(B, N // BN),
        in_specs=[
            pl.BlockSpec((1, C, BN), lambda i, j: (i, 0, j)),
            pl.BlockSpec((O, C), lambda i, j: (0, 0)),
            pl.BlockSpec((O,), lambda i, j: (0,)),
        ],
        out_specs=pl.BlockSpec((1, O, BN), lambda i, j: (i, 0, j)),
        out_shape=jax.ShapeDtypeStruct((B, O, N), jnp.float32),
    )(x, w, b)
    return out


def _c1d(x, p, act=True):
    y = jnp.einsum('oc,bcn->bon', p['w'], x) + p['b'][None, :, None]
    return jax.nn.relu(y) if act else y


def _ga_edgeconv(p, k, xyz, fea=None):
    if fea is None:
        fea = xyz
    idx = _knn(fea, fea, k)
    g = _get_edge_point(xyz, xyz, idx)
    f = _get_edge_feature(fea, fea, idx)
    g = _c2d(g, p['mg1'])
    g = _c2d(g, p['mg2'])
    f = _c2d(f, p['mf1'])
    f = _c2d(f, p['mf2'])
    f = _c2d(f, p['mf3'])
    return jnp.max(g * f, axis=-1)


def _sa_module(p, k, xyz, xyz_s, fea, fea_s):
    idx = _knn(xyz, xyz_s, k)
    g = _get_edge_point(xyz, xyz_s, idx)
    f = _get_edge_feature(fea, fea_s, idx)
    g = _c2d(g, p['mg'])
    f = _c2d(f, p['mf'])
    return jnp.max(g * f, axis=-1)


def kernel(xyz, xyz_s, params):
    K1, K2 = 16, 4
    xyz = jnp.transpose(xyz, (0, 2, 1))
    xyz_s = jnp.transpose(xyz_s, (0, 2, 1))
    fea1 = _ga_edgeconv(params['ec1'], K1, xyz)
    fea1_s = _ga_edgeconv(params['ec1'], K1, xyz_s)
    fea1 = _sa_module(params['sa1'], K2, xyz, xyz_s, fea1, fea1_s)
    fea2 = _ga_edgeconv(params['ec2'], K1, xyz, fea1)
    fea2_s = _ga_edgeconv(params['ec2s'], K1, xyz_s, fea1_s)
    fea2 = _sa_module(params['sa2'], K2, xyz, xyz_s, fea2, fea2_s)
    fea = _c1d_pallas(fea2, params['c1'])
    fea = _c1d_pallas(fea, params['c2'])
    g = jnp.max(fea, axis=-1, keepdims=True)
    g = jnp.broadcast_to(g, (g.shape[0], g.shape[1], xyz.shape[2]))
    fea = jnp.concatenate([fea1, fea2, g], axis=1)
    fea = _c1d_pallas(fea, params['c3'])
    fea = _c1d_pallas(fea, params['c4'])
    fea = _c1d_pallas(fea, params['c5'])
    fea = _c1d_pallas(fea, params['c6'], act=False)
    return fea


# speedup vs baseline: 13.6393x; 12.8439x over previous
import functools

import jax
import jax.numpy as jnp
from jax.experimental import pallas as pl
from jax.experimental.pallas import tpu as pltpu
from jax.experimental.pallas import tpu_sc as plsc

F32 = jnp.float32
NEG = float('-inf')


def _knn_body(K, BN, N, a_ref, bT_ref, a2_ref, b2_ref, o_ref):
    a = a_ref[0]
    bT = bT_ref[0]
    inner = -2.0 * jnp.dot(a, bT, preferred_element_type=F32)
    dist = -a2_ref[0] - inner - b2_ref[0]
    iota = jax.lax.broadcasted_iota(jnp.int32, (BN, N), 1)
    cols = []
    for _ in range(K):
        j = jnp.argmax(dist, axis=1)
        cols.append(j[:, None])
        dist = jnp.where(iota == j[:, None], NEG, dist)
    o_ref[0] = jnp.concatenate(cols, axis=1)


def _knn(a, bT, aT, K, BN=256):
    a2 = jnp.sum(aT * aT, axis=1)[:, :, None]
    b2 = jnp.sum(bT * bT, axis=1)[:, None, :]
    return _knn2(a, bT, a2, b2, K, BN)


def _knn2(a, bT, a2, b2, K, BN=256):
    Bc, N, Cp = a.shape
    return pl.pallas_call(
        functools.partial(_knn_body, K, BN, N),
        grid=(Bc, N // BN),
        in_specs=[
            pl.BlockSpec((1, BN, Cp), lambda b, i: (b, i, 0)),
            pl.BlockSpec((1, Cp, N), lambda b, i: (b, 0, 0)),
            pl.BlockSpec((1, BN, 1), lambda b, i: (b, i, 0)),
            pl.BlockSpec((1, 1, N), lambda b, i: (b, 0, 0)),
        ],
        out_specs=pl.BlockSpec((1, BN, K), lambda b, i: (b, i, 0)),
        out_shape=jax.ShapeDtypeStruct((Bc, N, K), jnp.int32),
    )(a, bT, a2, b2)


def _sc_gather(data, idx_flat):
    R, W = data.shape
    E = idx_flat.shape[1]
    GW = 128
    mesh = plsc.VectorSubcoreMesh(core_axis_name='core', subcore_axis_name='subcore')

    @functools.partial(
        pl.kernel,
        out_type=jax.ShapeDtypeStruct((E, W), data.dtype),
        mesh=mesh,
    )
    def gather_kernel(x_hbm, i_hbm, o_hbm):
        def body(i_vmem, o_vmem):
            pltpu.sync_copy(x_hbm.at[i_vmem.at[0]], o_vmem)

        pltpu.emit_pipeline(
            body,
            grid=(E // GW,),
            in_specs=[pl.BlockSpec((1, GW), lambda i: (0, i))],
            out_specs=[pl.BlockSpec((GW, W), lambda i: (i, 0))],
            core_axis_name=('core', 'subcore'),
            dimension_semantics=(pltpu.PARALLEL,),
        )(i_hbm, o_hbm)

    return gather_kernel(data, idx_flat)


def _gather_rows(rows, idx):
    Bc, N, W = rows.shape
    K = idx.shape[2]
    idxT = jnp.transpose(idx, (0, 2, 1))
    flat = (idxT + jnp.arange(Bc, dtype=jnp.int32)[:, None, None] * N).reshape(1, -1)
    out = _sc_gather(rows.reshape(Bc * N, W), flat)
    return out.reshape(Bc, K, N, W)


BF16 = jnp.bfloat16


def _bdot(x, w):
    return jnp.dot(x.astype(BF16), w.astype(BF16), preferred_element_type=F32)


def _ec_body(K, Cp, deep, g_ref, fs_ref, xs_ref, wfs_ref, wfn_ref, bf_ref,
             wd_ref, wx_ref, wk_ref, wr_ref, bg_ref, *rest):
    if deep:
        (wf2_ref, bf2_ref, wf3_ref, bf3_ref, wg2_ref, bg2_ref, o_ref) = rest
    else:
        (o_ref,) = rest
    fself = fs_ref[0]
    xs = xs_ref[0]
    fs = _bdot(fself, wfs_ref[0]) + bf_ref[0]
    gs = _bdot(xs, wx_ref[0]) + bg_ref[0]
    wdb = wd_ref[0].astype(BF16).astype(F32)
    acc = jnp.full(fs.shape, NEG, F32)
    for k in range(K):
        nb = g_ref[0, k]
        fn = nb[:, 0:Cp]
        xn = nb[:, Cp:Cp + 8]
        xr = xs - xn
        d2 = jnp.sum(xr * xr, axis=1, keepdims=True)
        d = jnp.sqrt(d2 + 1e-12)
        db = d.astype(BF16).astype(F32)
        f1 = jnp.maximum(fs + _bdot(fn, wfn_ref[0]), 0.0)
        g1 = jnp.maximum(gs + db * wdb + _bdot(xn, wk_ref[0])
                         + _bdot(xr, wr_ref[0]), 0.0)
        if deep:
            f2 = jnp.maximum(_bdot(f1, wf2_ref[0]) + bf2_ref[0], 0.0)
            f3 = jnp.maximum(_bdot(f2, wf3_ref[0]) + bf3_ref[0], 0.0)
            g2 = jnp.maximum(_bdot(g1, wg2_ref[0]) + bg2_ref[0], 0.0)
            h = g2 * f3
        else:
            h = g1 * f1
        acc = jnp.maximum(acc, h)
    o_ref[0] = acc


def _edgeconv(gath, fea_self, xyz_self, wpack, deep, BN=256):
    Bc, K, N, W = gath.shape
    Cp = fea_self.shape[2]
    S = wpack[0].shape[0]
    O = wpack[0].shape[2]
    per = Bc // S
    wspec2 = lambda d0, d1: pl.BlockSpec((1, d0, d1), lambda b, i: (b // per, 0, 0))
    in_specs = [
        pl.BlockSpec((1, K, BN, W), lambda b, i: (b, 0, i, 0)),
        pl.BlockSpec((1, BN, Cp), lambda b, i: (b, i, 0)),
        pl.BlockSpec((1, BN, 8), lambda b, i: (b, i, 0)),
        wspec2(Cp, O), wspec2(Cp, O), wspec2(1, O),
        wspec2(1, O), wspec2(8, O), wspec2(8, O), wspec2(8, O), wspec2(1, O),
    ]
    if deep:
        in_specs += [wspec2(O, O), wspec2(1, O), wspec2(O, O), wspec2(1, O),
                     wspec2(O, O), wspec2(1, O)]
    return pl.pallas_call(
        functools.partial(_ec_body, K, Cp, deep),
        grid=(Bc, N // BN),
        in_specs=in_specs,
        out_specs=pl.BlockSpec((1, BN, O), lambda b, i: (b, i, 0)),
        out_shape=jax.ShapeDtypeStruct((Bc, N, O), F32),
    )(gath, fea_self, xyz_self, *wpack)


def _lin_body(relu, x_ref, w_ref, b_ref, o_ref):
    y = _bdot(x_ref[0], w_ref[...]) + b_ref[...]
    if relu:
        y = jnp.maximum(y, 0.0)
    o_ref[0] = y


def _linear(x, w, b, relu=True, BN=512):
    Bc, M, C = x.shape
    O = w.shape[1]
    BN = min(BN, M)
    return pl.pallas_call(
        functools.partial(_lin_body, relu),
        grid=(Bc, M // BN),
        in_specs=[
            pl.BlockSpec((1, BN, C), lambda bt, i: (bt, i, 0)),
            pl.BlockSpec((C, O), lambda bt, i: (0, 0)),
            pl.BlockSpec((1, O), lambda bt, i: (0, 0)),
        ],
        out_specs=pl.BlockSpec((1, BN, O), lambda bt, i: (bt, i, 0)),
        out_shape=jax.ShapeDtypeStruct((Bc, M, O), F32),
    )(x, w, b)


def _lin_pool_body(x_ref, w_ref, b_ref, o_ref, g_ref):
    i = pl.program_id(1)
    y = jnp.maximum(_bdot(x_ref[0], w_ref[...]) + b_ref[...], 0.0)
    o_ref[0] = y
    m = jnp.max(y, axis=0, keepdims=True)

    @pl.when(i == 0)
    def _():
        g_ref[0] = m

    @pl.when(i != 0)
    def _():
        g_ref[0] = jnp.maximum(g_ref[0], m)


def _linear_pool(x, w, b, BN=512):
    Bc, M, C = x.shape
    O = w.shape[1]
    return pl.pallas_call(
        _lin_pool_body,
        grid=(Bc, M // BN),
        in_specs=[
            pl.BlockSpec((1, BN, C), lambda bt, i: (bt, i, 0)),
            pl.BlockSpec((C, O), lambda bt, i: (0, 0)),
            pl.BlockSpec((1, O), lambda bt, i: (0, 0)),
        ],
        out_specs=[
            pl.BlockSpec((1, BN, O), lambda bt, i: (bt, i, 0)),
            pl.BlockSpec((1, 1, O), lambda bt, i: (bt, 0, 0)),
        ],
        out_shape=[
            jax.ShapeDtypeStruct((Bc, M, O), F32),
            jax.ShapeDtypeStruct((Bc, 1, O), F32),
        ],
    )(x, w, b)


def _c3_body(f1_ref, f2_ref, gp_ref, w1_ref, w2_ref, o_ref):
    y = _bdot(f1_ref[0], w1_ref[...]) + _bdot(f2_ref[0], w2_ref[...]) + gp_ref[0]
    o_ref[0] = jnp.maximum(y, 0.0)


def _c3(fea1, fea2, gpart, w1, w2, BN=512):
    Bc, M, _ = fea1.shape
    O = w1.shape[1]
    return pl.pallas_call(
        _c3_body,
        grid=(Bc, M // BN),
        in_specs=[
            pl.BlockSpec((1, BN, fea1.shape[2]), lambda bt, i: (bt, i, 0)),
            pl.BlockSpec((1, BN, fea2.shape[2]), lambda bt, i: (bt, i, 0)),
            pl.BlockSpec((1, 1, O), lambda bt, i: (bt, 0, 0)),
            pl.BlockSpec((fea1.shape[2], O), lambda bt, i: (0, 0)),
            pl.BlockSpec((fea2.shape[2], O), lambda bt, i: (0, 0)),
        ],
        out_specs=pl.BlockSpec((1, BN, O), lambda bt, i: (bt, i, 0)),
        out_shape=jax.ShapeDtypeStruct((Bc, M, O), F32),
    )(fea1, fea2, gpart, w1, w2)


def _split_f(p, Cp, C, S_stack):
    w = p['w']
    ws = jnp.zeros((Cp, w.shape[0]), F32).at[:C].set(w[:, :C].T)
    wn = jnp.zeros((Cp, w.shape[0]), F32).at[:C].set(w[:, C:].T)
    return ws, wn, p['b'][None, :]


def _split_g(p):
    w = p['w']
    wd = w[:, 0][None, :]
    z8 = jnp.zeros((8, w.shape[0]), F32)
    wx = z8.at[:3].set(w[:, 1:4].T)
    wk = z8.at[:3].set(w[:, 4:7].T)
    wr = z8.at[:3].set(w[:, 7:10].T)
    return wd, wx, wk, wr, p['b'][None, :]


def _pack_shallow(ps, Cp, C):
    fs, fn, bf = zip(*[_split_f(p['mf' if 'mf' in p else 'mf1'], Cp, C, None)
                       for p in ps])
    gg = [_split_g(p['mg' if 'mg' in p else 'mg1']) for p in ps]
    wd, wx, wk, wr, bg = zip(*gg)
    st = lambda xs: jnp.stack(list(xs))
    return [st(fs), st(fn), st(bf), st(wd), st(wx), st(wk), st(wr), st(bg)]


def _pack_deep(ps, Cp, C):
    base = _pack_shallow(ps, Cp, C)
    st = lambda xs: jnp.stack(xs)
    for m in ('mf2', 'mf3', 'mg2'):
        base.append(st([p[m]['w'].T for p in ps]))
        base.append(st([p[m]['b'][None, :] for p in ps]))
    return base


def kernel(xyz, xyz_s, params):
    B, N, _ = xyz.shape
    K1, K2 = 16, 4
    pad8 = lambda x: jnp.pad(x, ((0, 0), (0, 0), (0, 5)))
    xyz8, xyzs8 = pad8(xyz), pad8(xyz_s)
    cat8 = jnp.concatenate([xyz8, xyzs8], 0)
    cat8T = jnp.transpose(cat8, (0, 2, 1))

    idx1 = _knn(cat8, cat8T, cat8T, K1)
    rows1 = jnp.concatenate([cat8, cat8, jnp.zeros((2 * B, N, 112), F32)], 2)
    g1 = _gather_rows(rows1, idx1)
    wp1 = _pack_deep([params['ec1']], 8, 3)
    fea1c = _edgeconv(g1, cat8, cat8, wp1, deep=True)
    fea1, fea1_s = fea1c[:B], fea1c[B:]

    idxs = _knn(xyz8, jnp.transpose(xyzs8, (0, 2, 1)),
                jnp.transpose(xyz8, (0, 2, 1)), K2)

    z88 = jnp.zeros((B, N, 88), F32)
    rows = jnp.concatenate([fea1_s, xyzs8, z88], 2)
    gs1 = _gather_rows(rows, idxs)
    wps1 = _pack_shallow([params['sa1']], 32, 32)
    fea1n = _edgeconv(gs1, fea1, xyz8, wps1, deep=False)

    feacat = jnp.concatenate([fea1n, fea1_s], 0)
    fcT = jnp.transpose(feacat, (0, 2, 1))
    idx2 = _knn(feacat, fcT, fcT, K1)
    rows = jnp.concatenate([feacat, cat8, jnp.zeros((2 * B, N, 88), F32)], 2)
    g2 = _gather_rows(rows, idx2)
    wp2 = _pack_deep([params['ec2'], params['ec2s']], 32, 32)
    fea2c = _edgeconv(g2, feacat, cat8, wp2, deep=True)
    fea2, fea2_s = fea2c[:B], fea2c[B:]

    rows = jnp.concatenate([fea2_s, xyzs8, jnp.zeros((B, N, 120), F32)], 2)
    gsa2 = _gather_rows(rows, idxs)
    wps2 = _pack_shallow([params['sa2']], 128, 128)
    fea2n = _edgeconv(gsa2, fea2, xyz8, wps2, deep=False)

    h = _linear(fea2n, params['c1']['w'].T, params['c1']['b'][None, :])
    h, gmax = _linear_pool(h, params['c2']['w'].T, params['c2']['b'][None, :])
    W3, b3 = params['c3']['w'], params['c3']['b']
    gpart = _linear(gmax, W3[:, 288:].T, b3[None, :], relu=False, BN=1)
    y = _c3(fea1n, fea2n, gpart, W3[:, :32].T, W3[:, 32:288].T)
    y = _linear(y, params['c4']['w'].T, params['c4']['b'][None, :])
    y = _linear(y, params['c5']['w'].T, params['c5']['b'][None, :])
    y = _linear(y, params['c6']['w'].T, params['c6']['b'][None, :], relu=False)
    return jnp.transpose(y, (0, 2, 1))


# speedup vs baseline: 14.7154x; 1.0789x over previous
import functools

import jax
import jax.numpy as jnp
from jax.experimental import pallas as pl
from jax.experimental.pallas import tpu as pltpu
from jax.experimental.pallas import tpu_sc as plsc

F32 = jnp.float32
NEG = float('-inf')


def _knn_body(K, BN, N, a_ref, bT_ref, a2_ref, b2_ref, o_ref):
    a = a_ref[0]
    bT = bT_ref[0]
    inner = -2.0 * jnp.dot(a, bT, preferred_element_type=F32)
    dist = -a2_ref[0] - inner - b2_ref[0]
    iota = jax.lax.broadcasted_iota(jnp.int32, (BN, N), 1)
    cols = []
    for _ in range(K):
        j = jnp.argmax(dist, axis=1)
        cols.append(j[:, None])
        dist = jnp.where(iota == j[:, None], NEG, dist)
    o_ref[0] = jnp.concatenate(cols, axis=1)


def _knn(a, bT, aT, K, BN=512):
    a2 = jnp.sum(aT * aT, axis=1)[:, :, None]
    b2 = jnp.sum(bT * bT, axis=1)[:, None, :]
    return _knn2(a, bT, a2, b2, K, BN)


def _knn2(a, bT, a2, b2, K, BN=512):
    Bc, N, Cp = a.shape
    return pl.pallas_call(
        functools.partial(_knn_body, K, BN, N),
        grid=(Bc, N // BN),
        in_specs=[
            pl.BlockSpec((1, BN, Cp), lambda b, i: (b, i, 0)),
            pl.BlockSpec((1, Cp, N), lambda b, i: (b, 0, 0)),
            pl.BlockSpec((1, BN, 1), lambda b, i: (b, i, 0)),
            pl.BlockSpec((1, 1, N), lambda b, i: (b, 0, 0)),
        ],
        out_specs=pl.BlockSpec((1, BN, K), lambda b, i: (b, i, 0)),
        out_shape=jax.ShapeDtypeStruct((Bc, N, K), jnp.int32),
    )(a, bT, a2, b2)


def _sc_gather(data, idx_flat):
    R, W = data.shape
    E = idx_flat.shape[1]
    GW = 128
    mesh = plsc.VectorSubcoreMesh(core_axis_name='core', subcore_axis_name='subcore')

    @functools.partial(
        pl.kernel,
        out_type=jax.ShapeDtypeStruct((E, W), data.dtype),
        mesh=mesh,
    )
    def gather_kernel(x_hbm, i_hbm, o_hbm):
        def body(i_vmem, o_vmem):
            pltpu.sync_copy(x_hbm.at[i_vmem.at[0]], o_vmem)

        pltpu.emit_pipeline(
            body,
            grid=(E // GW,),
            in_specs=[pl.BlockSpec((1, GW), lambda i: (0, i))],
            out_specs=[pl.BlockSpec((GW, W), lambda i: (i, 0))],
            core_axis_name=('core', 'subcore'),
            dimension_semantics=(pltpu.PARALLEL,),
        )(i_hbm, o_hbm)

    return gather_kernel(data, idx_flat)


def _gather_rows(rows, idx):
    Bc, N, W = rows.shape
    K = idx.shape[2]
    idxT = jnp.transpose(idx, (0, 2, 1))
    flat = (idxT + jnp.arange(Bc, dtype=jnp.int32)[:, None, None] * N).reshape(1, -1)
    out = _sc_gather(rows.reshape(Bc * N, W), flat)
    return out.reshape(Bc, K, N, W)


BF16 = jnp.bfloat16


def _bdot(x, w):
    return jnp.dot(x.astype(BF16), w.astype(BF16), preferred_element_type=F32)


def _ec_body(K, Cp, deep, g_ref, fs_ref, xs_ref, wfs_ref, wfn_ref, bf_ref,
             wx_ref, wg_ref, bg_ref, *rest):
    if deep:
        (wf2_ref, bf2_ref, wf3_ref, bf3_ref, wg2_ref, bg2_ref, o_ref) = rest
    else:
        (o_ref,) = rest
    fself = fs_ref[0]
    xs = xs_ref[0]
    fs = _bdot(fself, wfs_ref[0]) + bf_ref[0]
    gs = _bdot(xs, wx_ref[0]) + bg_ref[0]
    acc = jnp.full(fs.shape, NEG, F32)
    for k in range(K):
        nb = g_ref[0, k]
        fn = nb[:, 0:Cp]
        xn = nb[:, Cp:Cp + 8]
        xr = xs - xn
        d2 = jnp.sum(xr * xr, axis=1, keepdims=True)
        d = jnp.sqrt(d2 + 1e-12)
        e = jnp.concatenate([jnp.pad(d, ((0, 0), (0, 7))), xn, xr], axis=1)
        f1 = jnp.maximum(fs + _bdot(fn, wfn_ref[0]), 0.0)
        g1 = jnp.maximum(gs + _bdot(e, wg_ref[0]), 0.0)
        if deep:
            f2 = jnp.maximum(_bdot(f1, wf2_ref[0]) + bf2_ref[0], 0.0)
            f3 = jnp.maximum(_bdot(f2, wf3_ref[0]) + bf3_ref[0], 0.0)
            g2 = jnp.maximum(_bdot(g1, wg2_ref[0]) + bg2_ref[0], 0.0)
            h = g2 * f3
        else:
            h = g1 * f1
        acc = jnp.maximum(acc, h)
    o_ref[0] = acc


def _edgeconv(gath, fea_self, xyz_self, wpack, deep, BN=256):
    Bc, K, N, W = gath.shape
    Cp = fea_self.shape[2]
    S = wpack[0].shape[0]
    O = wpack[0].shape[2]
    per = Bc // S
    wspec2 = lambda d0, d1: pl.BlockSpec((1, d0, d1), lambda b, i: (b // per, 0, 0))
    in_specs = [
        pl.BlockSpec((1, K, BN, W), lambda b, i: (b, 0, i, 0)),
        pl.BlockSpec((1, BN, Cp), lambda b, i: (b, i, 0)),
        pl.BlockSpec((1, BN, 8), lambda b, i: (b, i, 0)),
        wspec2(Cp, O), wspec2(Cp, O), wspec2(1, O),
        wspec2(8, O), wspec2(24, O), wspec2(1, O),
    ]
    if deep:
        in_specs += [wspec2(O, O), wspec2(1, O), wspec2(O, O), wspec2(1, O),
                     wspec2(O, O), wspec2(1, O)]
    return pl.pallas_call(
        functools.partial(_ec_body, K, Cp, deep),
        grid=(Bc, N // BN),
        in_specs=in_specs,
        out_specs=pl.BlockSpec((1, BN, O), lambda b, i: (b, i, 0)),
        out_shape=jax.ShapeDtypeStruct((Bc, N, O), F32),
    )(gath, fea_self, xyz_self, *wpack)


def _lin_body(relu, x_ref, w_ref, b_ref, o_ref):
    y = _bdot(x_ref[0], w_ref[...]) + b_ref[...]
    if relu:
        y = jnp.maximum(y, 0.0)
    o_ref[0] = y


def _linear(x, w, b, relu=True, BN=512):
    Bc, M, C = x.shape
    O = w.shape[1]
    BN = min(BN, M)
    return pl.pallas_call(
        functools.partial(_lin_body, relu),
        grid=(Bc, M // BN),
        in_specs=[
            pl.BlockSpec((1, BN, C), lambda bt, i: (bt, i, 0)),
            pl.BlockSpec((C, O), lambda bt, i: (0, 0)),
            pl.BlockSpec((1, O), lambda bt, i: (0, 0)),
        ],
        out_specs=pl.BlockSpec((1, BN, O), lambda bt, i: (bt, i, 0)),
        out_shape=jax.ShapeDtypeStruct((Bc, M, O), F32),
    )(x, w, b)


def _lin_pool_body(x_ref, w_ref, b_ref, o_ref, g_ref):
    i = pl.program_id(1)
    y = jnp.maximum(_bdot(x_ref[0], w_ref[...]) + b_ref[...], 0.0)
    o_ref[0] = y
    m = jnp.max(y, axis=0, keepdims=True)

    @pl.when(i == 0)
    def _():
        g_ref[0] = m

    @pl.when(i != 0)
    def _():
        g_ref[0] = jnp.maximum(g_ref[0], m)


def _linear_pool(x, w, b, BN=512):
    Bc, M, C = x.shape
    O = w.shape[1]
    return pl.pallas_call(
        _lin_pool_body,
        grid=(Bc, M // BN),
        in_specs=[
            pl.BlockSpec((1, BN, C), lambda bt, i: (bt, i, 0)),
            pl.BlockSpec((C, O), lambda bt, i: (0, 0)),
            pl.BlockSpec((1, O), lambda bt, i: (0, 0)),
        ],
        out_specs=[
            pl.BlockSpec((1, BN, O), lambda bt, i: (bt, i, 0)),
            pl.BlockSpec((1, 1, O), lambda bt, i: (bt, 0, 0)),
        ],
        out_shape=[
            jax.ShapeDtypeStruct((Bc, M, O), F32),
            jax.ShapeDtypeStruct((Bc, 1, O), F32),
        ],
    )(x, w, b)


def _c3_body(f1_ref, f2_ref, gp_ref, w1_ref, w2_ref, o_ref):
    y = _bdot(f1_ref[0], w1_ref[...]) + _bdot(f2_ref[0], w2_ref[...]) + gp_ref[0]
    o_ref[0] = jnp.maximum(y, 0.0)


def _c3(fea1, fea2, gpart, w1, w2, BN=512):
    Bc, M, _ = fea1.shape
    O = w1.shape[1]
    return pl.pallas_call(
        _c3_body,
        grid=(Bc, M // BN),
        in_specs=[
            pl.BlockSpec((1, BN, fea1.shape[2]), lambda bt, i: (bt, i, 0)),
            pl.BlockSpec((1, BN, fea2.shape[2]), lambda bt, i: (bt, i, 0)),
            pl.BlockSpec((1, 1, O), lambda bt, i: (bt, 0, 0)),
            pl.BlockSpec((fea1.shape[2], O), lambda bt, i: (0, 0)),
            pl.BlockSpec((fea2.shape[2], O), lambda bt, i: (0, 0)),
        ],
        out_specs=pl.BlockSpec((1, BN, O), lambda bt, i: (bt, i, 0)),
        out_shape=jax.ShapeDtypeStruct((Bc, M, O), F32),
    )(fea1, fea2, gpart, w1, w2)


def _split_f(p, Cp, C, S_stack):
    w = p['w']
    ws = jnp.zeros((Cp, w.shape[0]), F32).at[:C].set(w[:, :C].T)
    wn = jnp.zeros((Cp, w.shape[0]), F32).at[:C].set(w[:, C:].T)
    return ws, wn, p['b'][None, :]


def _split_g(p):
    w = p['w']
    O = w.shape[0]
    wx = jnp.zeros((8, O), F32).at[:3].set(w[:, 1:4].T)
    wg = jnp.zeros((24, O), F32)
    wg = wg.at[0].set(w[:, 0])
    wg = wg.at[8:11].set(w[:, 4:7].T)
    wg = wg.at[16:19].set(w[:, 7:10].T)
    return wx, wg, p['b'][None, :]


def _pack_shallow(ps, Cp, C):
    fs, fn, bf = zip(*[_split_f(p['mf' if 'mf' in p else 'mf1'], Cp, C, None)
                       for p in ps])
    gg = [_split_g(p['mg' if 'mg' in p else 'mg1']) for p in ps]
    wx, wg, bg = zip(*gg)
    st = lambda xs: jnp.stack(list(xs))
    return [st(fs), st(fn), st(bf), st(wx), st(wg), st(bg)]


def _pack_deep(ps, Cp, C):
    base = _pack_shallow(ps, Cp, C)
    st = lambda xs: jnp.stack(xs)
    for m in ('mf2', 'mf3', 'mg2'):
        base.append(st([p[m]['w'].T for p in ps]))
        base.append(st([p[m]['b'][None, :] for p in ps]))
    return base


def kernel(xyz, xyz_s, params):
    B, N, _ = xyz.shape
    K1, K2 = 16, 4
    pad8 = lambda x: jnp.pad(x, ((0, 0), (0, 0), (0, 5)))
    xyz8, xyzs8 = pad8(xyz), pad8(xyz_s)
    xyz8T = jnp.transpose(xyz8, (0, 2, 1))
    xyzs8T = jnp.transpose(xyzs8, (0, 2, 1))

    wp1 = _pack_deep([params['ec1']], 8, 3)
    z112 = jnp.zeros((B, N, 112), F32)

    def ec1(x8, x8T):
        idx = _knn(x8, x8T, x8T, K1)
        rows = jnp.concatenate([x8, x8, z112], 2)
        g = _gather_rows(rows, idx)
        return _edgeconv(g, x8, x8, wp1, deep=True)

    fea1 = ec1(xyz8, xyz8T)
    fea1_s = ec1(xyzs8, xyzs8T)

    idxs = _knn(xyz8, xyzs8T, xyz8T, K2)

    z88 = jnp.zeros((B, N, 88), F32)

    def ec2(p, fea, x8):
        feaT = jnp.transpose(fea, (0, 2, 1))
        idx = _knn(fea, feaT, feaT, K1)
        rows = jnp.concatenate([fea, x8, z88], 2)
        g = _gather_rows(rows, idx)
        wp = _pack_deep([p], 32, 32)
        return _edgeconv(g, fea, x8, wp, deep=True)

    fea2_s = ec2(params['ec2s'], fea1_s, xyzs8)

    rows = jnp.concatenate([fea1_s, xyzs8, z88], 2)
    gs1 = _gather_rows(rows, idxs)
    wps1 = _pack_shallow([params['sa1']], 32, 32)
    fea1n = _edgeconv(gs1, fea1, xyz8, wps1, deep=False)

    fea2 = ec2(params['ec2'], fea1n, xyz8)

    rows = jnp.concatenate([fea2_s, xyzs8, jnp.zeros((B, N, 120), F32)], 2)
    gsa2 = _gather_rows(rows, idxs)
    wps2 = _pack_shallow([params['sa2']], 128, 128)
    fea2n = _edgeconv(gsa2, fea2, xyz8, wps2, deep=False)

    h = _linear(fea2n, params['c1']['w'].T, params['c1']['b'][None, :])
    h, gmax = _linear_pool(h, params['c2']['w'].T, params['c2']['b'][None, :])
    W3, b3 = params['c3']['w'], params['c3']['b']
    gpart = _linear(gmax, W3[:, 288:].T, b3[None, :], relu=False, BN=1)
    y = _c3(fea1n, fea2n, gpart, W3[:, :32].T, W3[:, 32:288].T)
    y = _linear(y, params['c4']['w'].T, params['c4']['b'][None, :])
    y = _linear(y, params['c5']['w'].T, params['c5']['b'][None, :])
    y = _linear(y, params['c6']['w'].T, params['c6']['b'][None, :], relu=False)
    return jnp.transpose(y, (0, 2, 1))
